# baseline (speedup 1.0000x reference)
import functools

import jax
import jax.numpy as jnp
from jax import lax
from jax.experimental import pallas as pl
from jax.experimental.pallas import tpu as pltpu
from jax.experimental.pallas import tpu_sc as plsc

VOCAB = 100000
MAX_POS = 2048
EMBED = 768
BATCH = 16
SEQ = 2048

_NC, _NS, _L = 2, 16, 16
_NW = _NC * _NS
_POS_PER_W = SEQ // _NW
_NB = 8


def _body(ids_hbm, token_hbm, pos_hbm, out_hbm,
          idx_v, oidx_v, posb, rows, gsem, ssem, psem):
    wid = lax.axis_index("s") * _NC + lax.axis_index("c")
    s_base = wid * _POS_PER_W

    ids_src = ids_hbm.at[
        pl.ds(pl.multiple_of(s_base * BATCH, 8), _POS_PER_W * BATCH)]
    pltpu.sync_copy(ids_src, idx_v)

    def fire_gather(s, p):
        pltpu.async_copy(
            token_hbm.at[idx_v.at[pl.ds(s * BATCH, BATCH)]], rows[p], gsem[p])
        pltpu.async_copy(
            pos_hbm.at[pl.ds((s_base + s) * EMBED, EMBED)], posb[p], psem[p])

    def wait_gather(s, p):
        pltpu.make_async_copy(
            token_hbm.at[idx_v.at[pl.ds(s * BATCH, BATCH)]], rows[p],
            gsem[p]).wait()
        pltpu.make_async_copy(
            pos_hbm.at[pl.ds((s_base + s) * EMBED, EMBED)], posb[p],
            psem[p]).wait()

    def fire_store(s, p):
        pltpu.async_copy(rows[p], out_hbm.at[oidx_v.at[s]], ssem[p])

    def wait_store(s, p):
        pltpu.make_async_copy(
            rows[p], out_hbm.at[oidx_v.at[s]], ssem[p]).wait()

    def add_chunk(s, p):
        pvals = [posb[p][pl.ds(k * _L, _L)] for k in range(EMBED // _L)]

        def add_row(j, c):
            for k in range(EMBED // _L):
                sl = pl.ds(k * _L, _L)
                rows[p][j, sl] = rows[p][j, sl] + pvals[k]
            return c

        lax.fori_loop(0, BATCH, add_row, 0)

    for t in range(_NB - 1):
        fire_gather(t, t)

    bvec = lax.iota(jnp.int32, _L) * SEQ + s_base

    def gen_oidx(s, c):
        oidx_v[s, :] = bvec + s
        return c

    lax.fori_loop(0, _POS_PER_W, gen_oidx, 0)

    def body(i, carry):
        for u in range(_NB):
            s = i * _NB + u
            wait_gather(s, u)
            add_chunk(s, u)
            fire_store(s, u)
            pn = (u + _NB - 1) % _NB
            if u == 0:
                @pl.when(i > 0)
                def _():
                    wait_store(s - 1, pn)

                fire_gather(s + _NB - 1, pn)
            else:
                @pl.when(i < _POS_PER_W // _NB - 1)
                def _():
                    wait_store(s - 1, pn)
                    fire_gather(s + _NB - 1, pn)
        return carry

    lax.fori_loop(0, _POS_PER_W // _NB, body, 0)

    for u in range(_NB):
        wait_store(_POS_PER_W - _NB + u, u)


@functools.cache
def _build():
    return pl.kernel(
        _body,
        out_type=jax.ShapeDtypeStruct((BATCH * SEQ, EMBED), jnp.float32),
        mesh=plsc.VectorSubcoreMesh(
            core_axis_name="c", subcore_axis_name="s",
            num_cores=_NC, num_subcores=_NS,
        ),
        scratch_types=[
            pltpu.VMEM((_POS_PER_W * BATCH,), jnp.int32),
            pltpu.VMEM((_POS_PER_W, _L), jnp.int32),
            [pltpu.VMEM((EMBED,), jnp.float32) for _ in range(_NB)],
            [pltpu.VMEM((BATCH, EMBED), jnp.float32) for _ in range(_NB)],
            [pltpu.SemaphoreType.DMA for _ in range(_NB)],
            [pltpu.SemaphoreType.DMA for _ in range(_NB)],
            [pltpu.SemaphoreType.DMA for _ in range(_NB)],
        ],
    )


def kernel(input_ids, token_table, pos_table):
    ids_t = input_ids.astype(jnp.int32).T.reshape(-1)
    out = _build()(ids_t, token_table, pos_table.reshape(-1))
    return out.reshape(BATCH, SEQ, EMBED)


# speedup vs baseline: 1.0565x; 1.0565x over previous
import functools

import jax
import jax.numpy as jnp
from jax import lax
from jax.experimental import pallas as pl
from jax.experimental.pallas import tpu as pltpu
from jax.experimental.pallas import tpu_sc as plsc

VOCAB = 100000
MAX_POS = 2048
EMBED = 768
BATCH = 16
SEQ = 2048

_NC, _NS, _L = 2, 16, 16
_NW = _NC * _NS
_POS_PER_W = SEQ // _NW
_NB = 4


def _body(ids_hbm, token_hbm, pos_hbm, out_hbm,
          idx_v, oidx_v, pos_v, rows, gsem, ssem):
    wid = lax.axis_index("s") * _NC + lax.axis_index("c")
    s_base = wid * _POS_PER_W

    ids_src = ids_hbm.at[
        pl.ds(pl.multiple_of(s_base * BATCH, 8), _POS_PER_W * BATCH)]
    pltpu.sync_copy(ids_src, idx_v)
    pos_src = pos_hbm.at[pl.ds(s_base, _POS_PER_W)]
    pos_cp = pltpu.async_copy(pos_src, pos_v, gsem[_NB - 1])

    def fire_gather(s, p):
        pltpu.async_copy(
            token_hbm.at[idx_v.at[pl.ds(s * BATCH, BATCH)]], rows[p], gsem[p])

    def wait_gather(s, p):
        pltpu.make_async_copy(
            token_hbm.at[idx_v.at[pl.ds(s * BATCH, BATCH)]], rows[p],
            gsem[p]).wait()

    def fire_store(s, p):
        pltpu.async_copy(rows[p], out_hbm.at[oidx_v.at[s]], ssem[p])

    def wait_store(s, p):
        pltpu.make_async_copy(
            rows[p], out_hbm.at[oidx_v.at[s]], ssem[p]).wait()

    def add_chunk(s, p):
        pvals = [pos_v[s, pl.ds(k * _L, _L)] for k in range(EMBED // _L)]

        def add_row(j, c):
            for k in range(EMBED // _L):
                sl = pl.ds(k * _L, _L)
                rows[p][j, sl] = rows[p][j, sl] + pvals[k]
            return c

        lax.fori_loop(0, BATCH, add_row, 0)

    for t in range(_NB - 1):
        fire_gather(t, t)

    bvec = lax.iota(jnp.int32, _L) * SEQ + s_base

    def gen_oidx(s, c):
        oidx_v[s, :] = bvec + s
        return c

    lax.fori_loop(0, _POS_PER_W, gen_oidx, 0)
    pos_cp.wait()

    def body(i, carry):
        for u in range(_NB):
            s = i * _NB + u
            wait_gather(s, u)
            add_chunk(s, u)
            fire_store(s, u)
            pn = (u + _NB - 1) % _NB
            if u == 0:
                @pl.when(i > 0)
                def _():
                    wait_store(s - 1, pn)

                fire_gather(s + _NB - 1, pn)
            else:
                @pl.when(i < _POS_PER_W // _NB - 1)
                def _():
                    wait_store(s - 1, pn)
                    fire_gather(s + _NB - 1, pn)
        return carry

    lax.fori_loop(0, _POS_PER_W // _NB, body, 0)

    for u in range(_NB):
        wait_store(_POS_PER_W - _NB + u, u)


@functools.cache
def _build():
    return pl.kernel(
        _body,
        out_type=jax.ShapeDtypeStruct((BATCH * SEQ, EMBED), jnp.float32),
        mesh=plsc.VectorSubcoreMesh(
            core_axis_name="c", subcore_axis_name="s",
            num_cores=_NC, num_subcores=_NS,
        ),
        scratch_types=[
            pltpu.VMEM((_POS_PER_W * BATCH,), jnp.int32),
            pltpu.VMEM((_POS_PER_W, _L), jnp.int32),
            pltpu.VMEM((_POS_PER_W, EMBED), jnp.float32),
            [pltpu.VMEM((BATCH, EMBED), jnp.float32) for _ in range(_NB)],
            [pltpu.SemaphoreType.DMA for _ in range(_NB)],
            [pltpu.SemaphoreType.DMA for _ in range(_NB)],
        ],
    )


def kernel(input_ids, token_table, pos_table):
    ids_t = input_ids.astype(jnp.int32).T.reshape(-1)
    out = _build()(ids_t, token_table, pos_table)
    return out.reshape(BATCH, SEQ, EMBED)
